# Initial kernel scaffold; baseline (speedup 1.0000x reference)
import jax
import jax.numpy as jnp
from jax.experimental import pallas as pl


def kernel(feat, edge_index, edge_type, W):
    raise NotImplementedError("write your pallas kernel here")


# speedup vs baseline: 11.2469x; 11.2469x over previous
import jax
import jax.numpy as jnp
from jax import lax
from jax.experimental import pallas as pl
from jax.experimental.pallas import tpu as pltpu
from jax.experimental.pallas import tpu_sc as plsc

_N = 10000
_E = 320000
_D = 128
_R = 8
_DH = _D // 2

_NC = 2
_NS = 16
_CHUNK = 128
_CPT = 160
_EPAD = _NS * _CPT * _CHUNK
_NACC = 10240
_ZROWS = _NACC // _NS
_BATCH = 16


def _h_body(feat_ref, w_ref, h_ref):
    h_ref[0, 0] = lax.dot_general(
        feat_ref[...], w_ref[0],
        (((1,), (1,)), ((), ())),
        preferred_element_type=jnp.float32,
    )


_h_call = pl.pallas_call(
    _h_body,
    grid=(_NC, _R),
    in_specs=[
        pl.BlockSpec((_N, _D), lambda c, r: (0, 0)),
        pl.BlockSpec((1, _DH, _D), lambda c, r: (r, c, 0)),
    ],
    out_specs=pl.BlockSpec((1, 1, _N, _DH), lambda c, r: (c, r, 0, 0)),
    out_shape=jax.ShapeDtypeStruct((_NC, _R, _N, _DH), jnp.float32),
)


def _sc_body(h_hbm, src_hbm, et_hbm, dst_hbm, out_hbm,
             src_b, et_b, dst_v, g_v, rows_v, stage_v, y_sh, sem):
    c = lax.axis_index("c")
    s = lax.axis_index("s")
    base = s * _CPT

    pltpu.sync_copy(dst_hbm.at[pl.ds(base, _CPT)], dst_v)

    zvec = jnp.zeros((16,), jnp.float32)

    def _zrow(i, carry):
        for k in range(_DH // 16):
            stage_v[i, pl.ds(k * 16, 16)] = zvec
        return carry

    lax.fori_loop(0, _CHUNK, _zrow, 0)
    for m in range(_ZROWS // _CHUNK):
        pltpu.sync_copy(stage_v, y_sh.at[pl.ds(s * _ZROWS + m * _CHUNK, _CHUNK)])

    goff = c * (_R * _N)

    for b in range(_CPT // _BATCH):
        pltpu.sync_copy(src_hbm.at[pl.ds(base + b * _BATCH, _BATCH)], src_b)
        pltpu.sync_copy(et_hbm.at[pl.ds(base + b * _BATCH, _BATCH)], et_b)

        def _gidx(j, carry, b=b):
            for k in range(_CHUNK // 16):
                sl = pl.ds(k * 16, 16)
                g_v[b * _BATCH + j, sl] = et_b[j, sl] * _N + src_b[j, sl] + goff
            return carry

        lax.fori_loop(0, _BATCH, _gidx, 0)

    plsc.subcore_barrier()

    def _edge_chunk(j, carry):
        pltpu.async_copy(h_hbm.at[g_v.at[j]], rows_v, sem).wait()
        pltpu.sync_copy(rows_v, y_sh.at[dst_v.at[j]], add=True)
        return carry

    lax.fori_loop(0, _CPT, _edge_chunk, 0)

    plsc.subcore_barrier()

    for m in range(_ZROWS // _CHUNK):
        r0 = s * _ZROWS + m * _CHUNK
        pltpu.sync_copy(y_sh.at[pl.ds(r0, _CHUNK)], stage_v)
        pltpu.sync_copy(stage_v, out_hbm.at[c, pl.ds(r0, _CHUNK)])


_sc_call = pl.kernel(
    _sc_body,
    out_type=jax.ShapeDtypeStruct((_NC, _NACC, _DH), jnp.float32),
    mesh=plsc.VectorSubcoreMesh(core_axis_name="c", subcore_axis_name="s"),
    compiler_params=pltpu.CompilerParams(use_tc_tiling_on_sc=False),
    scratch_types=[
        pltpu.VMEM((_BATCH, _CHUNK), jnp.int32),
        pltpu.VMEM((_BATCH, _CHUNK), jnp.int32),
        pltpu.VMEM((_CPT, _CHUNK), jnp.int32),
        pltpu.VMEM((_CPT, _CHUNK), jnp.int32),
        pltpu.VMEM((_CHUNK, _DH), jnp.float32),
        pltpu.VMEM((_CHUNK, _DH), jnp.float32),
        pltpu.VMEM_SHARED((_NACC, _DH), jnp.float32),
        pltpu.SemaphoreType.DMA,
    ],
)


def _cat_body(p_ref, o_ref):
    o_ref[:, 0:_DH] = p_ref[0]
    o_ref[:, _DH:_D] = p_ref[1]


_cat_call = pl.pallas_call(
    _cat_body,
    grid=(10,),
    in_specs=[pl.BlockSpec((_NC, _N // 10, _DH), lambda i: (0, i, 0))],
    out_specs=pl.BlockSpec((_N // 10, _D), lambda i: (i, 0)),
    out_shape=jax.ShapeDtypeStruct((_N, _D), jnp.float32),
)


def kernel(feat, edge_index, edge_type, W):
    src = edge_index[0]
    dst = edge_index[1]
    h = _h_call(feat, W).reshape(_NC * _R * _N, _DH)
    pad = _EPAD - _E
    src_p = jnp.concatenate([src, jnp.zeros((pad,), jnp.int32)])
    et_p = jnp.concatenate([edge_type, jnp.zeros((pad,), jnp.int32)])
    dst_p = jnp.concatenate([dst, jnp.full((pad,), _N, jnp.int32)])
    shape2d = (_NS * _CPT, _CHUNK)
    parts = _sc_call(h,
                     src_p.reshape(shape2d),
                     et_p.reshape(shape2d),
                     dst_p.reshape(shape2d))
    return _cat_call(parts)
